# Initial kernel scaffold; baseline (speedup 1.0000x reference)
import functools

import jax
import jax.numpy as jnp
from jax import lax
from jax.experimental import pallas as pl
from jax.experimental.pallas import tpu as pltpu
from jax.experimental.pallas import tpu_sc as plsc

N = 10000
E = 320000
D_IN = 128
D_H = 64
MLP_H = 128

NC = 2
NS = 16
NW = NC * NS
BLK = 128
N_PAD = 10240
RPT = N_PAD // NS
EPT_BLKS = 79
EPT = EPT_BLKS * BLK
E_PAD = NW * EPT
CNT_W = 16


def _sc_aggregate(with_counts):
    mesh = plsc.VectorSubcoreMesh(core_axis_name="c", subcore_axis_name="s")
    out_type = [jax.ShapeDtypeStruct((NC * N_PAD, D_H), jnp.float32)]
    scratch = [
        pltpu.VMEM((BLK,), jnp.int32),
        pltpu.VMEM((BLK,), jnp.int32),
        pltpu.VMEM((BLK, D_H), jnp.float32),
        pltpu.VMEM_SHARED((N_PAD, D_H), jnp.float32),
        pltpu.SemaphoreType.DMA,
    ]
    if with_counts:
        out_type.append(jax.ShapeDtypeStruct((NC * N_PAD, CNT_W), jnp.float32))
        scratch += [
            pltpu.VMEM((BLK, CNT_W), jnp.float32),
            pltpu.VMEM_SHARED((N_PAD, CNT_W), jnp.float32),
        ]

    def body(y_hbm, src_hbm, dst_hbm, z64_hbm, z16_hbm, ones_hbm,
             acc_out, *rest):
        if with_counts:
            cnt_out, src_v, dst_v, rows_v, acc_sh, sem, ones_v, cnt_sh = rest
        else:
            src_v, dst_v, rows_v, acc_sh, sem = rest
        cid = lax.axis_index("c")
        sid = lax.axis_index("s")
        wid = cid * NS + sid

        ro = pl.multiple_of(sid * RPT, RPT)
        pltpu.sync_copy(z64_hbm.at[pl.ds(ro, RPT)], acc_sh.at[pl.ds(ro, RPT)])
        if with_counts:
            pltpu.sync_copy(z16_hbm.at[pl.ds(ro, RPT)],
                            cnt_sh.at[pl.ds(ro, RPT)])
            pltpu.sync_copy(ones_hbm, ones_v)
        plsc.subcore_barrier()

        ebase = wid * EPT

        def blk_body(i, carry):
            off = pl.multiple_of(ebase + i * BLK, BLK)
            pltpu.sync_copy(src_hbm.at[pl.ds(off, BLK)], src_v)
            pltpu.sync_copy(dst_hbm.at[pl.ds(off, BLK)], dst_v)
            pltpu.async_copy(y_hbm.at[src_v], rows_v, sem).wait()
            pltpu.sync_copy(rows_v, acc_sh.at[dst_v], add=True)
            if with_counts:
                pltpu.sync_copy(ones_v, cnt_sh.at[dst_v], add=True)
            return carry

        lax.fori_loop(0, EPT_BLKS, blk_body, 0)
        plsc.subcore_barrier()

        oo = pl.multiple_of(cid * N_PAD + ro, RPT)
        pltpu.sync_copy(acc_sh.at[pl.ds(ro, RPT)], acc_out.at[pl.ds(oo, RPT)])
        if with_counts:
            pltpu.sync_copy(cnt_sh.at[pl.ds(ro, RPT)],
                            cnt_out.at[pl.ds(oo, RPT)])

    return pl.kernel(body, out_type=out_type, mesh=mesh,
                     scratch_types=scratch)


def _tc_pre(x, Wl, Wr):
    n, _ = x.shape
    dh = Wl.shape[1]

    def body(x_ref, wl_ref, wr_ref, ypad_ref, z_ref):
        xv = x_ref[...]
        y = jnp.dot(xv, wl_ref[...], preferred_element_type=jnp.float32)
        ypad_ref[0:n, :] = y
        ypad_ref[n:N_PAD, :] = jnp.zeros((N_PAD - n, dh), jnp.float32)
        z_ref[...] = jnp.dot(xv, wr_ref[...],
                             preferred_element_type=jnp.float32)

    return pl.pallas_call(
        body,
        out_shape=[jax.ShapeDtypeStruct((N_PAD, dh), jnp.float32),
                   jax.ShapeDtypeStruct((n, dh), jnp.float32)],
    )(x, Wl, Wr)


def _tc_mid(acc, cnt, z, bl, g, be, Wl_next, Wr_next):
    def body(acc_ref, cnt_ref, z_ref, bl_ref, g_ref, be_ref, wl_ref, wr_ref,
             ypad_ref, znext_ref):
        agg = acc_ref[0:N, :] + acc_ref[N_PAD:N_PAD + N, :]
        c = cnt_ref[0:N, 0:1] + cnt_ref[N_PAD:N_PAD + N, 0:1]
        mean = agg / jnp.maximum(c, 1.0)
        t = mean + bl_ref[...] + z_ref[...]
        mu = jnp.mean(t, axis=-1, keepdims=True)
        var = jnp.mean((t - mu) ** 2, axis=-1, keepdims=True)
        h = (t - mu) / jnp.sqrt(var + 1e-5) * g_ref[...] + be_ref[...]
        h = h * jax.nn.sigmoid(h)
        y = jnp.dot(h, wl_ref[...], preferred_element_type=jnp.float32)
        ypad_ref[0:N, :] = y
        ypad_ref[N:N_PAD, :] = jnp.zeros((N_PAD - N, D_H), jnp.float32)
        znext_ref[...] = jnp.dot(h, wr_ref[...],
                                 preferred_element_type=jnp.float32)

    return pl.pallas_call(
        body,
        out_shape=[jax.ShapeDtypeStruct((N_PAD, D_H), jnp.float32),
                   jax.ShapeDtypeStruct((N, D_H), jnp.float32)],
    )(acc, cnt, z, bl, g, be, Wl_next, Wr_next)


def _tc_post(acc, cnt, z, bl, g, be, Wm1, bm1, Wm2, bm2):
    def body(acc_ref, cnt_ref, z_ref, bl_ref, g_ref, be_ref,
             wm1_ref, bm1_ref, wm2_ref, bm2_ref, out_ref):
        agg = acc_ref[0:N, :] + acc_ref[N_PAD:N_PAD + N, :]
        c = cnt_ref[0:N, 0:1] + cnt_ref[N_PAD:N_PAD + N, 0:1]
        mean = agg / jnp.maximum(c, 1.0)
        t = mean + bl_ref[...] + z_ref[...]
        mu = jnp.mean(t, axis=-1, keepdims=True)
        var = jnp.mean((t - mu) ** 2, axis=-1, keepdims=True)
        h = (t - mu) / jnp.sqrt(var + 1e-5) * g_ref[...] + be_ref[...]
        h = h * jax.nn.sigmoid(h)
        m = jnp.dot(h, wm1_ref[...], preferred_element_type=jnp.float32)
        m = jnp.maximum(m + bm1_ref[...], 0.0)
        out_ref[...] = (jnp.dot(m, wm2_ref[...],
                                preferred_element_type=jnp.float32)
                        + bm2_ref[...])

    return pl.pallas_call(
        body,
        out_shape=jax.ShapeDtypeStruct((N, 1), jnp.float32),
    )(acc, cnt, z, bl, g, be, Wm1, bm1, Wm2, bm2)


@jax.jit
def kernel(x, edge_index, W0l, b0l, W0r, g0, be0, W1l, b1l, W1r, g1, be1,
           Wm1, bm1, Wm2, bm2):
    src = edge_index[0]
    dst = edge_index[1]
    pad = E_PAD - E
    src_p = jnp.concatenate([src, jnp.full((pad,), N, jnp.int32)])
    dst_p = jnp.concatenate([dst, jnp.full((pad,), N, jnp.int32)])

    z64 = jnp.zeros((N_PAD, D_H), jnp.float32)
    z16 = jnp.zeros((N_PAD, CNT_W), jnp.float32)
    ones = jnp.ones((BLK, CNT_W), jnp.float32)

    b0l_r = b0l.reshape(1, D_H)
    g0_r = g0.reshape(1, D_H)
    be0_r = be0.reshape(1, D_H)
    b1l_r = b1l.reshape(1, D_H)
    g1_r = g1.reshape(1, D_H)
    be1_r = be1.reshape(1, D_H)
    bm1_r = bm1.reshape(1, MLP_H)
    bm2_r = bm2.reshape(1, 1)

    agg0_fn = _sc_aggregate(with_counts=True)
    agg1_fn = _sc_aggregate(with_counts=False)

    y0_pad, z0 = _tc_pre(x, W0l, W0r)
    acc0, cnt = agg0_fn(y0_pad, src_p, dst_p, z64, z16, ones)
    y1_pad, z1 = _tc_mid(acc0, cnt, z0, b0l_r, g0_r, be0_r, W1l, W1r)

    acc1 = agg1_fn(y1_pad, src_p, dst_p, z64, z16, ones)

    return _tc_post(acc1, cnt, z1, b1l_r, g1_r, be1_r,
                    Wm1, bm1_r, Wm2, bm2_r)


# speedup vs baseline: 5.8442x; 5.8442x over previous
import functools

import jax
import jax.numpy as jnp
from jax import lax
from jax.experimental import pallas as pl
from jax.experimental.pallas import tpu as pltpu
from jax.experimental.pallas import tpu_sc as plsc

N = 10000
E = 320000
D_IN = 128
D_H = 64
MLP_H = 128

NC = 2
NS = 16
NW = NC * NS
BLK = 128
N_PAD = 10240
RPT = N_PAD // NS
EPT_BLKS = 79
EPT = EPT_BLKS * BLK
E_PAD = NW * EPT
CNT_W = 16


def _sc_aggregate(with_counts):
    mesh = plsc.VectorSubcoreMesh(core_axis_name="c", subcore_axis_name="s")
    out_type = [jax.ShapeDtypeStruct((NC * N_PAD, D_H), jnp.float32)]
    scratch = [
        pltpu.VMEM((BLK,), jnp.int32),
        pltpu.VMEM((BLK,), jnp.int32),
        pltpu.VMEM((BLK, D_H), jnp.float32),
        pltpu.VMEM_SHARED((N_PAD, D_H), jnp.float32),
        pltpu.SemaphoreType.DMA,
    ]
    if with_counts:
        out_type.append(jax.ShapeDtypeStruct((NC * N_PAD, CNT_W), jnp.float32))
        scratch += [
            pltpu.VMEM((BLK, CNT_W), jnp.float32),
            pltpu.VMEM_SHARED((N_PAD, CNT_W), jnp.float32),
        ]

    def body(y_hbm, src_hbm, dst_hbm, z64_hbm, z16_hbm, ones_hbm,
             acc_out, *rest):
        if with_counts:
            cnt_out, src_v, dst_v, rows_v, acc_sh, sem, ones_v, cnt_sh = rest
        else:
            src_v, dst_v, rows_v, acc_sh, sem = rest
        cid = lax.axis_index("c")
        sid = lax.axis_index("s")
        wid = cid * NS + sid

        ro = pl.multiple_of(sid * RPT, RPT)
        pltpu.sync_copy(z64_hbm.at[pl.ds(ro, RPT)], acc_sh.at[pl.ds(ro, RPT)])
        if with_counts:
            pltpu.sync_copy(z16_hbm.at[pl.ds(ro, RPT)],
                            cnt_sh.at[pl.ds(ro, RPT)])
            pltpu.sync_copy(ones_hbm, ones_v)
        plsc.subcore_barrier()

        ebase = wid * EPT

        def blk_body(i, carry):
            off = pl.multiple_of(ebase + i * BLK, BLK)
            pltpu.sync_copy(src_hbm.at[pl.ds(off, BLK)], src_v)
            pltpu.sync_copy(dst_hbm.at[pl.ds(off, BLK)], dst_v)
            pltpu.async_copy(y_hbm.at[src_v], rows_v, sem).wait()
            pltpu.sync_copy(rows_v, acc_sh.at[dst_v], add=True)
            if with_counts:
                pltpu.sync_copy(ones_v, cnt_sh.at[dst_v], add=True)
            return carry

        lax.fori_loop(0, EPT_BLKS, blk_body, 0)
        plsc.subcore_barrier()

        oo = pl.multiple_of(cid * N_PAD + ro, RPT)
        pltpu.sync_copy(acc_sh.at[pl.ds(ro, RPT)], acc_out.at[pl.ds(oo, RPT)])
        if with_counts:
            pltpu.sync_copy(cnt_sh.at[pl.ds(ro, RPT)],
                            cnt_out.at[pl.ds(oo, RPT)])

    return pl.kernel(body, out_type=out_type, mesh=mesh,
                     scratch_types=scratch,
                     compiler_params=pltpu.CompilerParams(
                         use_tc_tiling_on_sc=False))


def _tc_pre(x, Wl, Wr):
    n, _ = x.shape
    dh = Wl.shape[1]

    def body(x_ref, wl_ref, wr_ref, ypad_ref, z_ref):
        xv = x_ref[...]
        y = jnp.dot(xv, wl_ref[...], preferred_element_type=jnp.float32)
        ypad_ref[0:n, :] = y
        ypad_ref[n:N_PAD, :] = jnp.zeros((N_PAD - n, dh), jnp.float32)
        z_ref[...] = jnp.dot(xv, wr_ref[...],
                             preferred_element_type=jnp.float32)

    return pl.pallas_call(
        body,
        out_shape=[jax.ShapeDtypeStruct((N_PAD, dh), jnp.float32),
                   jax.ShapeDtypeStruct((n, dh), jnp.float32)],
    )(x, Wl, Wr)


def _tc_mid(acc, cnt, z, bl, g, be, Wl_next, Wr_next):
    def body(acc_ref, cnt_ref, z_ref, bl_ref, g_ref, be_ref, wl_ref, wr_ref,
             ypad_ref, znext_ref):
        agg = acc_ref[0:N, :] + acc_ref[N_PAD:N_PAD + N, :]
        c = cnt_ref[0:N, 0:1] + cnt_ref[N_PAD:N_PAD + N, 0:1]
        mean = agg / jnp.maximum(c, 1.0)
        t = mean + bl_ref[...] + z_ref[...]
        mu = jnp.mean(t, axis=-1, keepdims=True)
        var = jnp.mean((t - mu) ** 2, axis=-1, keepdims=True)
        h = (t - mu) / jnp.sqrt(var + 1e-5) * g_ref[...] + be_ref[...]
        h = h * jax.nn.sigmoid(h)
        y = jnp.dot(h, wl_ref[...], preferred_element_type=jnp.float32)
        ypad_ref[0:N, :] = y
        ypad_ref[N:N_PAD, :] = jnp.zeros((N_PAD - N, D_H), jnp.float32)
        znext_ref[...] = jnp.dot(h, wr_ref[...],
                                 preferred_element_type=jnp.float32)

    return pl.pallas_call(
        body,
        out_shape=[jax.ShapeDtypeStruct((N_PAD, D_H), jnp.float32),
                   jax.ShapeDtypeStruct((N, D_H), jnp.float32)],
    )(acc, cnt, z, bl, g, be, Wl_next, Wr_next)


def _tc_post(acc, cnt, z, bl, g, be, Wm1, bm1, Wm2, bm2):
    def body(acc_ref, cnt_ref, z_ref, bl_ref, g_ref, be_ref,
             wm1_ref, bm1_ref, wm2_ref, bm2_ref, out_ref):
        agg = acc_ref[0:N, :] + acc_ref[N_PAD:N_PAD + N, :]
        c = cnt_ref[0:N, 0:1] + cnt_ref[N_PAD:N_PAD + N, 0:1]
        mean = agg / jnp.maximum(c, 1.0)
        t = mean + bl_ref[...] + z_ref[...]
        mu = jnp.mean(t, axis=-1, keepdims=True)
        var = jnp.mean((t - mu) ** 2, axis=-1, keepdims=True)
        h = (t - mu) / jnp.sqrt(var + 1e-5) * g_ref[...] + be_ref[...]
        h = h * jax.nn.sigmoid(h)
        m = jnp.dot(h, wm1_ref[...], preferred_element_type=jnp.float32)
        m = jnp.maximum(m + bm1_ref[...], 0.0)
        out_ref[...] = (jnp.dot(m, wm2_ref[...],
                                preferred_element_type=jnp.float32)
                        + bm2_ref[...])

    return pl.pallas_call(
        body,
        out_shape=jax.ShapeDtypeStruct((N, 1), jnp.float32),
    )(acc, cnt, z, bl, g, be, Wm1, bm1, Wm2, bm2)


@jax.jit
def kernel(x, edge_index, W0l, b0l, W0r, g0, be0, W1l, b1l, W1r, g1, be1,
           Wm1, bm1, Wm2, bm2):
    src = edge_index[0]
    dst = edge_index[1]
    pad = E_PAD - E
    src_p = jnp.concatenate([src, jnp.full((pad,), N, jnp.int32)])
    dst_p = jnp.concatenate([dst, jnp.full((pad,), N, jnp.int32)])

    z64 = jnp.zeros((N_PAD, D_H), jnp.float32)
    z16 = jnp.zeros((N_PAD, CNT_W), jnp.float32)
    ones = jnp.ones((BLK, CNT_W), jnp.float32)

    b0l_r = b0l.reshape(1, D_H)
    g0_r = g0.reshape(1, D_H)
    be0_r = be0.reshape(1, D_H)
    b1l_r = b1l.reshape(1, D_H)
    g1_r = g1.reshape(1, D_H)
    be1_r = be1.reshape(1, D_H)
    bm1_r = bm1.reshape(1, MLP_H)
    bm2_r = bm2.reshape(1, 1)

    agg0_fn = _sc_aggregate(with_counts=True)
    agg1_fn = _sc_aggregate(with_counts=False)

    y0_pad, z0 = _tc_pre(x, W0l, W0r)
    acc0, cnt = agg0_fn(y0_pad, src_p, dst_p, z64, z16, ones)
    y1_pad, z1 = _tc_mid(acc0, cnt, z0, b0l_r, g0_r, be0_r, W1l, W1r)

    acc1, = agg1_fn(y1_pad, src_p, dst_p, z64, z16, ones)

    return _tc_post(acc1, cnt, z1, b1l_r, g1_r, be1_r,
                    Wm1, bm1_r, Wm2, bm2_r)


# speedup vs baseline: 5.9122x; 1.0116x over previous
import functools

import jax
import jax.numpy as jnp
from jax import lax
from jax.experimental import pallas as pl
from jax.experimental.pallas import tpu as pltpu
from jax.experimental.pallas import tpu_sc as plsc

N = 10000
E = 320000
D_IN = 128
D_H = 64
MLP_H = 128

NC = 2
NS = 16
NW = NC * NS
BLK = 128
N_PAD = 10240
RPT = N_PAD // NS
EPT_BLKS = 80
EPT = EPT_BLKS * BLK
E_PAD = NW * EPT
NBLK_TOT = E_PAD // BLK
CNT_W = 16
NBUF = 4


def _sc_aggregate(with_counts):
    mesh = plsc.VectorSubcoreMesh(core_axis_name="c", subcore_axis_name="s")
    out_type = [jax.ShapeDtypeStruct((NC * N_PAD, D_H), jnp.float32)]
    scratch = [
        pltpu.VMEM((EPT_BLKS, BLK), jnp.int32),
        pltpu.VMEM((EPT_BLKS, BLK), jnp.int32),
        [pltpu.VMEM((BLK, D_H), jnp.float32) for _ in range(NBUF)],
        [pltpu.SemaphoreType.DMA for _ in range(NBUF)],
        [pltpu.SemaphoreType.DMA for _ in range(NBUF)],
        pltpu.VMEM_SHARED((N_PAD, D_H), jnp.float32),
    ]
    if with_counts:
        out_type.append(jax.ShapeDtypeStruct((NC * N_PAD, CNT_W), jnp.float32))
        scratch += [
            pltpu.VMEM((BLK, CNT_W), jnp.float32),
            pltpu.VMEM_SHARED((N_PAD, CNT_W), jnp.float32),
            [pltpu.SemaphoreType.DMA for _ in range(NBUF)],
        ]

    def body(y_hbm, src_hbm, dst_hbm, z64_hbm, z16_hbm, ones_hbm,
             acc_out, *rest):
        if with_counts:
            (cnt_out, src_v, dst_v, rows, gsem, ssem, acc_sh,
             ones_v, cnt_sh, csem) = rest
        else:
            src_v, dst_v, rows, gsem, ssem, acc_sh = rest
        cid = lax.axis_index("c")
        sid = lax.axis_index("s")
        wid = cid * NS + sid

        ro = pl.multiple_of(sid * RPT, RPT)
        pltpu.sync_copy(z64_hbm.at[pl.ds(ro, RPT)], acc_sh.at[pl.ds(ro, RPT)])
        if with_counts:
            pltpu.sync_copy(z16_hbm.at[pl.ds(ro, RPT)],
                            cnt_sh.at[pl.ds(ro, RPT)])
            pltpu.sync_copy(ones_hbm, ones_v)
        bbase = pl.multiple_of(wid * EPT_BLKS, EPT_BLKS)
        pltpu.sync_copy(src_hbm.at[pl.ds(bbase, EPT_BLKS)], src_v)
        pltpu.sync_copy(dst_hbm.at[pl.ds(bbase, EPT_BLKS)], dst_v)
        plsc.subcore_barrier()

        def start_gather(blk, j):
            pltpu.async_copy(y_hbm.at[src_v.at[blk]], rows[j], gsem[j])

        def wait_gather(j):
            pltpu.make_async_copy(y_hbm.at[src_v.at[0]], rows[j],
                                  gsem[j]).wait()

        def start_scatter(blk, j):
            pltpu.async_copy(rows[j], acc_sh.at[dst_v.at[blk]], ssem[j],
                             add=True)
            if with_counts:
                pltpu.async_copy(ones_v, cnt_sh.at[dst_v.at[blk]], csem[j],
                                 add=True)

        def wait_scatter(j):
            pltpu.make_async_copy(rows[j], acc_sh.at[dst_v.at[0]],
                                  ssem[j]).wait()
            if with_counts:
                pltpu.make_async_copy(ones_v, cnt_sh.at[dst_v.at[0]],
                                      csem[j]).wait()

        for j in range(NBUF):
            start_gather(j, j)

        def step(k, carry):
            b = k * NBUF
            for j in range(NBUF):
                wait_gather(j)
                start_scatter(b + j, j)
            for j in range(NBUF):
                wait_scatter(j)
                start_gather(b + NBUF + j, j)
            return carry

        lax.fori_loop(0, EPT_BLKS // NBUF - 1, step, 0)

        tail = EPT_BLKS - NBUF
        for j in range(NBUF):
            wait_gather(j)
            start_scatter(tail + j, j)
        for j in range(NBUF):
            wait_scatter(j)

        plsc.subcore_barrier()

        oo = pl.multiple_of(cid * N_PAD + ro, RPT)
        pltpu.sync_copy(acc_sh.at[pl.ds(ro, RPT)], acc_out.at[pl.ds(oo, RPT)])
        if with_counts:
            pltpu.sync_copy(cnt_sh.at[pl.ds(ro, RPT)],
                            cnt_out.at[pl.ds(oo, RPT)])

    return pl.kernel(body, out_type=out_type, mesh=mesh,
                     scratch_types=scratch,
                     compiler_params=pltpu.CompilerParams(
                         use_tc_tiling_on_sc=False))


def _tc_pre(x, Wl, Wr):
    n, _ = x.shape
    dh = Wl.shape[1]

    def body(x_ref, wl_ref, wr_ref, ypad_ref, z_ref):
        xv = x_ref[...]
        y = jnp.dot(xv, wl_ref[...], preferred_element_type=jnp.float32)
        ypad_ref[0:n, :] = y
        ypad_ref[n:N_PAD, :] = jnp.zeros((N_PAD - n, dh), jnp.float32)
        z_ref[...] = jnp.dot(xv, wr_ref[...],
                             preferred_element_type=jnp.float32)

    return pl.pallas_call(
        body,
        out_shape=[jax.ShapeDtypeStruct((N_PAD, dh), jnp.float32),
                   jax.ShapeDtypeStruct((n, dh), jnp.float32)],
    )(x, Wl, Wr)


def _tc_mid(acc, cnt, z, bl, g, be, Wl_next, Wr_next):
    def body(acc_ref, cnt_ref, z_ref, bl_ref, g_ref, be_ref, wl_ref, wr_ref,
             ypad_ref, znext_ref):
        agg = acc_ref[0:N, :] + acc_ref[N_PAD:N_PAD + N, :]
        c = cnt_ref[0:N, 0:1] + cnt_ref[N_PAD:N_PAD + N, 0:1]
        mean = agg / jnp.maximum(c, 1.0)
        t = mean + bl_ref[...] + z_ref[...]
        mu = jnp.mean(t, axis=-1, keepdims=True)
        var = jnp.mean((t - mu) ** 2, axis=-1, keepdims=True)
        h = (t - mu) / jnp.sqrt(var + 1e-5) * g_ref[...] + be_ref[...]
        h = h * jax.nn.sigmoid(h)
        y = jnp.dot(h, wl_ref[...], preferred_element_type=jnp.float32)
        ypad_ref[0:N, :] = y
        ypad_ref[N:N_PAD, :] = jnp.zeros((N_PAD - N, D_H), jnp.float32)
        znext_ref[...] = jnp.dot(h, wr_ref[...],
                                 preferred_element_type=jnp.float32)

    return pl.pallas_call(
        body,
        out_shape=[jax.ShapeDtypeStruct((N_PAD, D_H), jnp.float32),
                   jax.ShapeDtypeStruct((N, D_H), jnp.float32)],
    )(acc, cnt, z, bl, g, be, Wl_next, Wr_next)


def _tc_post(acc, cnt, z, bl, g, be, Wm1, bm1, Wm2, bm2):
    def body(acc_ref, cnt_ref, z_ref, bl_ref, g_ref, be_ref,
             wm1_ref, bm1_ref, wm2_ref, bm2_ref, out_ref):
        agg = acc_ref[0:N, :] + acc_ref[N_PAD:N_PAD + N, :]
        c = cnt_ref[0:N, 0:1] + cnt_ref[N_PAD:N_PAD + N, 0:1]
        mean = agg / jnp.maximum(c, 1.0)
        t = mean + bl_ref[...] + z_ref[...]
        mu = jnp.mean(t, axis=-1, keepdims=True)
        var = jnp.mean((t - mu) ** 2, axis=-1, keepdims=True)
        h = (t - mu) / jnp.sqrt(var + 1e-5) * g_ref[...] + be_ref[...]
        h = h * jax.nn.sigmoid(h)
        m = jnp.dot(h, wm1_ref[...], preferred_element_type=jnp.float32)
        m = jnp.maximum(m + bm1_ref[...], 0.0)
        out_ref[...] = (jnp.dot(m, wm2_ref[...],
                                preferred_element_type=jnp.float32)
                        + bm2_ref[...])

    return pl.pallas_call(
        body,
        out_shape=jax.ShapeDtypeStruct((N, 1), jnp.float32),
    )(acc, cnt, z, bl, g, be, Wm1, bm1, Wm2, bm2)


@jax.jit
def kernel(x, edge_index, W0l, b0l, W0r, g0, be0, W1l, b1l, W1r, g1, be1,
           Wm1, bm1, Wm2, bm2):
    src = edge_index[0]
    dst = edge_index[1]
    pad = E_PAD - E
    src_p = jnp.concatenate([src, jnp.full((pad,), N, jnp.int32)])
    dst_p = jnp.concatenate([dst, jnp.full((pad,), N, jnp.int32)])
    src2d = src_p.reshape(NBLK_TOT, BLK)
    dst2d = dst_p.reshape(NBLK_TOT, BLK)

    z64 = jnp.zeros((N_PAD, D_H), jnp.float32)
    z16 = jnp.zeros((N_PAD, CNT_W), jnp.float32)
    ones = jnp.ones((BLK, CNT_W), jnp.float32)

    b0l_r = b0l.reshape(1, D_H)
    g0_r = g0.reshape(1, D_H)
    be0_r = be0.reshape(1, D_H)
    b1l_r = b1l.reshape(1, D_H)
    g1_r = g1.reshape(1, D_H)
    be1_r = be1.reshape(1, D_H)
    bm1_r = bm1.reshape(1, MLP_H)
    bm2_r = bm2.reshape(1, 1)

    agg0_fn = _sc_aggregate(with_counts=True)
    agg1_fn = _sc_aggregate(with_counts=False)

    y0_pad, z0 = _tc_pre(x, W0l, W0r)
    acc0, cnt = agg0_fn(y0_pad, src2d, dst2d, z64, z16, ones)
    y1_pad, z1 = _tc_mid(acc0, cnt, z0, b0l_r, g0_r, be0_r, W1l, W1r)

    acc1, = agg1_fn(y1_pad, src2d, dst2d, z64, z16, ones)

    return _tc_post(acc1, cnt, z1, b1l_r, g1_r, be1_r,
                    Wm1, bm1_r, Wm2, bm2_r)


# speedup vs baseline: 5.9236x; 1.0019x over previous
import functools

import jax
import jax.numpy as jnp
from jax import lax
from jax.experimental import pallas as pl
from jax.experimental.pallas import tpu as pltpu
from jax.experimental.pallas import tpu_sc as plsc

N = 10000
E = 320000
D_IN = 128
D_H = 64
MLP_H = 128

NC = 2
NS = 16
NW = NC * NS
BLK = 128
N_PAD = 10240
RPT = N_PAD // NS
EPT_BLKS = 80
EPT = EPT_BLKS * BLK
E_PAD = NW * EPT
NBLK_TOT = E_PAD // BLK
CNT_W = 16
NBUF = 4


def _sc_aggregate(with_counts):
    mesh = plsc.VectorSubcoreMesh(core_axis_name="c", subcore_axis_name="s")
    out_type = [jax.ShapeDtypeStruct((NC * N_PAD, D_H), jnp.float32)]
    scratch = [
        pltpu.VMEM((EPT_BLKS, BLK), jnp.int32),
        pltpu.VMEM((EPT_BLKS, BLK), jnp.int32),
        [pltpu.VMEM((BLK, D_H), jnp.float32) for _ in range(NBUF)],
        [pltpu.SemaphoreType.DMA for _ in range(NBUF)],
        [pltpu.SemaphoreType.DMA for _ in range(NBUF)],
        pltpu.VMEM_SHARED((N_PAD, D_H), jnp.float32),
    ]
    if with_counts:
        out_type.append(jax.ShapeDtypeStruct((NC * N_PAD, CNT_W), jnp.float32))
        scratch += [
            pltpu.VMEM((BLK, CNT_W), jnp.float32),
            pltpu.VMEM_SHARED((N_PAD, CNT_W), jnp.float32),
            [pltpu.SemaphoreType.DMA for _ in range(NBUF)],
        ]

    def body(y_hbm, src_hbm, dst_hbm, z64_hbm, z16_hbm, ones_hbm,
             acc_out, *rest):
        if with_counts:
            (cnt_out, src_v, dst_v, rows, gsem, ssem, acc_sh,
             ones_v, cnt_sh, csem) = rest
        else:
            src_v, dst_v, rows, gsem, ssem, acc_sh = rest
        cid = lax.axis_index("c")
        sid = lax.axis_index("s")
        wid = cid * NS + sid

        ro = pl.multiple_of(sid * RPT, RPT)
        pltpu.sync_copy(z64_hbm.at[pl.ds(ro, RPT)], acc_sh.at[pl.ds(ro, RPT)])
        if with_counts:
            pltpu.sync_copy(z16_hbm.at[pl.ds(ro, RPT)],
                            cnt_sh.at[pl.ds(ro, RPT)])
            pltpu.sync_copy(ones_hbm, ones_v)
        bbase = pl.multiple_of(wid * EPT_BLKS, EPT_BLKS)
        pltpu.sync_copy(src_hbm.at[pl.ds(bbase, EPT_BLKS)], src_v)
        pltpu.sync_copy(dst_hbm.at[pl.ds(bbase, EPT_BLKS)], dst_v)
        plsc.subcore_barrier()

        def start_gather(blk, j):
            pltpu.async_copy(y_hbm.at[src_v.at[blk]], rows[j], gsem[j])

        def wait_gather(j):
            pltpu.make_async_copy(y_hbm.at[src_v.at[0]], rows[j],
                                  gsem[j]).wait()

        def start_scatter(blk, j):
            if True:
                return
            pltpu.async_copy(rows[j], acc_sh.at[dst_v.at[blk]], ssem[j],
                             add=True)
            if with_counts:
                pltpu.async_copy(ones_v, cnt_sh.at[dst_v.at[blk]], csem[j],
                                 add=True)

        def wait_scatter(j):
            if True:
                return
            pltpu.make_async_copy(rows[j], acc_sh.at[dst_v.at[0]],
                                  ssem[j]).wait()
            if with_counts:
                pltpu.make_async_copy(ones_v, cnt_sh.at[dst_v.at[0]],
                                      csem[j]).wait()

        for j in range(NBUF):
            start_gather(j, j)

        def step(k, carry):
            b = k * NBUF
            for j in range(NBUF):
                wait_gather(j)
                start_scatter(b + j, j)
            for j in range(NBUF):
                wait_scatter(j)
                start_gather(b + NBUF + j, j)
            return carry

        lax.fori_loop(0, EPT_BLKS // NBUF - 1, step, 0)

        tail = EPT_BLKS - NBUF
        for j in range(NBUF):
            wait_gather(j)
            start_scatter(tail + j, j)
        for j in range(NBUF):
            wait_scatter(j)

        plsc.subcore_barrier()

        oo = pl.multiple_of(cid * N_PAD + ro, RPT)
        pltpu.sync_copy(acc_sh.at[pl.ds(ro, RPT)], acc_out.at[pl.ds(oo, RPT)])
        if with_counts:
            pltpu.sync_copy(cnt_sh.at[pl.ds(ro, RPT)],
                            cnt_out.at[pl.ds(oo, RPT)])

    return pl.kernel(body, out_type=out_type, mesh=mesh,
                     scratch_types=scratch,
                     compiler_params=pltpu.CompilerParams(
                         use_tc_tiling_on_sc=False))


def _tc_pre(x, Wl, Wr):
    n, _ = x.shape
    dh = Wl.shape[1]

    def body(x_ref, wl_ref, wr_ref, ypad_ref, z_ref):
        xv = x_ref[...]
        y = jnp.dot(xv, wl_ref[...], preferred_element_type=jnp.float32)
        ypad_ref[0:n, :] = y
        ypad_ref[n:N_PAD, :] = jnp.zeros((N_PAD - n, dh), jnp.float32)
        z_ref[...] = jnp.dot(xv, wr_ref[...],
                             preferred_element_type=jnp.float32)

    return pl.pallas_call(
        body,
        out_shape=[jax.ShapeDtypeStruct((N_PAD, dh), jnp.float32),
                   jax.ShapeDtypeStruct((n, dh), jnp.float32)],
    )(x, Wl, Wr)


def _tc_mid(acc, cnt, z, bl, g, be, Wl_next, Wr_next):
    def body(acc_ref, cnt_ref, z_ref, bl_ref, g_ref, be_ref, wl_ref, wr_ref,
             ypad_ref, znext_ref):
        agg = acc_ref[0:N, :] + acc_ref[N_PAD:N_PAD + N, :]
        c = cnt_ref[0:N, 0:1] + cnt_ref[N_PAD:N_PAD + N, 0:1]
        mean = agg / jnp.maximum(c, 1.0)
        t = mean + bl_ref[...] + z_ref[...]
        mu = jnp.mean(t, axis=-1, keepdims=True)
        var = jnp.mean((t - mu) ** 2, axis=-1, keepdims=True)
        h = (t - mu) / jnp.sqrt(var + 1e-5) * g_ref[...] + be_ref[...]
        h = h * jax.nn.sigmoid(h)
        y = jnp.dot(h, wl_ref[...], preferred_element_type=jnp.float32)
        ypad_ref[0:N, :] = y
        ypad_ref[N:N_PAD, :] = jnp.zeros((N_PAD - N, D_H), jnp.float32)
        znext_ref[...] = jnp.dot(h, wr_ref[...],
                                 preferred_element_type=jnp.float32)

    return pl.pallas_call(
        body,
        out_shape=[jax.ShapeDtypeStruct((N_PAD, D_H), jnp.float32),
                   jax.ShapeDtypeStruct((N, D_H), jnp.float32)],
    )(acc, cnt, z, bl, g, be, Wl_next, Wr_next)


def _tc_post(acc, cnt, z, bl, g, be, Wm1, bm1, Wm2, bm2):
    def body(acc_ref, cnt_ref, z_ref, bl_ref, g_ref, be_ref,
             wm1_ref, bm1_ref, wm2_ref, bm2_ref, out_ref):
        agg = acc_ref[0:N, :] + acc_ref[N_PAD:N_PAD + N, :]
        c = cnt_ref[0:N, 0:1] + cnt_ref[N_PAD:N_PAD + N, 0:1]
        mean = agg / jnp.maximum(c, 1.0)
        t = mean + bl_ref[...] + z_ref[...]
        mu = jnp.mean(t, axis=-1, keepdims=True)
        var = jnp.mean((t - mu) ** 2, axis=-1, keepdims=True)
        h = (t - mu) / jnp.sqrt(var + 1e-5) * g_ref[...] + be_ref[...]
        h = h * jax.nn.sigmoid(h)
        m = jnp.dot(h, wm1_ref[...], preferred_element_type=jnp.float32)
        m = jnp.maximum(m + bm1_ref[...], 0.0)
        out_ref[...] = (jnp.dot(m, wm2_ref[...],
                                preferred_element_type=jnp.float32)
                        + bm2_ref[...])

    return pl.pallas_call(
        body,
        out_shape=jax.ShapeDtypeStruct((N, 1), jnp.float32),
    )(acc, cnt, z, bl, g, be, Wm1, bm1, Wm2, bm2)


@jax.jit
def kernel(x, edge_index, W0l, b0l, W0r, g0, be0, W1l, b1l, W1r, g1, be1,
           Wm1, bm1, Wm2, bm2):
    src = edge_index[0]
    dst = edge_index[1]
    pad = E_PAD - E
    src_p = jnp.concatenate([src, jnp.full((pad,), N, jnp.int32)])
    dst_p = jnp.concatenate([dst, jnp.full((pad,), N, jnp.int32)])
    src2d = src_p.reshape(NBLK_TOT, BLK)
    dst2d = dst_p.reshape(NBLK_TOT, BLK)

    z64 = jnp.zeros((N_PAD, D_H), jnp.float32)
    z16 = jnp.zeros((N_PAD, CNT_W), jnp.float32)
    ones = jnp.ones((BLK, CNT_W), jnp.float32)

    b0l_r = b0l.reshape(1, D_H)
    g0_r = g0.reshape(1, D_H)
    be0_r = be0.reshape(1, D_H)
    b1l_r = b1l.reshape(1, D_H)
    g1_r = g1.reshape(1, D_H)
    be1_r = be1.reshape(1, D_H)
    bm1_r = bm1.reshape(1, MLP_H)
    bm2_r = bm2.reshape(1, 1)

    agg0_fn = _sc_aggregate(with_counts=True)
    agg1_fn = _sc_aggregate(with_counts=False)

    y0_pad, z0 = _tc_pre(x, W0l, W0r)
    acc0, cnt = agg0_fn(y0_pad, src2d, dst2d, z64, z16, ones)
    y1_pad, z1 = _tc_mid(acc0, cnt, z0, b0l_r, g0_r, be0_r, W1l, W1r)

    acc1, = agg1_fn(y1_pad, src2d, dst2d, z64, z16, ones)

    return _tc_post(acc1, cnt, z1, b1l_r, g1_r, be1_r,
                    Wm1, bm1_r, Wm2, bm2_r)


# speedup vs baseline: 11.2241x; 1.8948x over previous
import functools

import jax
import jax.numpy as jnp
from jax import lax
from jax.experimental import pallas as pl
from jax.experimental.pallas import tpu as pltpu
from jax.experimental.pallas import tpu_sc as plsc

N = 10000
E = 320000
D_IN = 128
D_H = 64
MLP_H = 128

NC = 2
NS = 16
DH_C = D_H // NC
BLK = 128
N_PAD = 10016
RPT = N_PAD // NS
BPT = 160
HALF = BPT // 2
E_PAD = NS * BPT * BLK
NBLK_TOT = E_PAD // BLK
CNT_W = 16
NBUF = 4


def _sc_aggregate(with_counts):
    mesh = plsc.VectorSubcoreMesh(core_axis_name="c", subcore_axis_name="s")
    out_type = [jax.ShapeDtypeStruct((NC, N_PAD, DH_C), jnp.float32)]
    scratch = [
        pltpu.VMEM((BPT, BLK), jnp.int32),
        pltpu.VMEM((BPT, BLK), jnp.int32),
        [pltpu.VMEM((BLK, DH_C), jnp.float32) for _ in range(NBUF)],
        [pltpu.SemaphoreType.DMA for _ in range(NBUF)],
        [pltpu.SemaphoreType.DMA for _ in range(NBUF)],
        pltpu.VMEM_SHARED((N_PAD, DH_C), jnp.float32),
        pltpu.VMEM_SHARED((N_PAD, DH_C), jnp.float32),
    ]
    if with_counts:
        out_type.append(
            jax.ShapeDtypeStruct((NC, N_PAD, CNT_W), jnp.float32))
        scratch += [
            pltpu.VMEM((BLK, CNT_W), jnp.float32),
            pltpu.VMEM_SHARED((N_PAD, CNT_W), jnp.float32),
            [pltpu.SemaphoreType.DMA for _ in range(NBUF)],
        ]

    def body(y_hbm, src_hbm, dst_hbm, z32_hbm, z16_hbm, ones_hbm,
             acc_out, *rest):
        it = iter(rest)
        cnt_out = next(it) if with_counts else None
        src_v = next(it)
        dst_v = next(it)
        rows = next(it)
        gsem = next(it)
        ssem = next(it)
        acc_sh = next(it)
        y_sh = next(it)
        if with_counts:
            ones_v = next(it)
            cnt_sh = next(it)
            csem = next(it)
        cid = lax.axis_index("c")
        sid = lax.axis_index("s")

        ro = pl.multiple_of(sid * RPT, RPT)
        pltpu.sync_copy(y_hbm.at[cid, pl.ds(ro, RPT)], y_sh.at[pl.ds(ro, RPT)])
        pltpu.sync_copy(z32_hbm.at[pl.ds(ro, RPT)], acc_sh.at[pl.ds(ro, RPT)])
        if with_counts:
            pltpu.sync_copy(z16_hbm.at[pl.ds(ro, RPT)],
                            cnt_sh.at[pl.ds(ro, RPT)])
            pltpu.sync_copy(ones_hbm, ones_v)
        bbase = pl.multiple_of(sid * BPT, BPT)
        pltpu.sync_copy(src_hbm.at[pl.ds(bbase, BPT)], src_v)
        pltpu.sync_copy(dst_hbm.at[pl.ds(bbase, BPT)], dst_v)
        plsc.subcore_barrier()

        def start_gather(blk, j):
            pltpu.async_copy(y_sh.at[src_v.at[blk]], rows[j], gsem[j])

        def wait_gather(j):
            pltpu.make_async_copy(y_sh.at[src_v.at[0]], rows[j],
                                  gsem[j]).wait()

        def run_phase(base, count_core):
            do_cnt = with_counts and count_core is not None
            if do_cnt:
                on = cid == count_core

            def start_scatter(blk, j):
                pltpu.async_copy(rows[j], acc_sh.at[dst_v.at[blk]], ssem[j],
                                 add=True)
                if do_cnt:
                    @pl.when(on)
                    def _():
                        pltpu.async_copy(ones_v, cnt_sh.at[dst_v.at[blk]],
                                         csem[j], add=True)

            def wait_scatter(j):
                pltpu.make_async_copy(rows[j], acc_sh.at[dst_v.at[0]],
                                      ssem[j]).wait()
                if do_cnt:
                    @pl.when(on)
                    def _():
                        pltpu.make_async_copy(ones_v, cnt_sh.at[dst_v.at[0]],
                                              csem[j]).wait()

            for j in range(NBUF):
                start_gather(base + j, j)

            def step(k, carry):
                b = base + k * NBUF
                for j in range(NBUF):
                    wait_gather(j)
                    start_scatter(b + j, j)
                for j in range(NBUF):
                    wait_scatter(j)
                    start_gather(b + NBUF + j, j)
                return carry

            lax.fori_loop(0, HALF // NBUF - 1, step, 0)

            tail = base + HALF - NBUF
            for j in range(NBUF):
                wait_gather(j)
                start_scatter(tail + j, j)
            for j in range(NBUF):
                wait_scatter(j)

        run_phase(0, 0 if with_counts else None)
        run_phase(HALF, 1 if with_counts else None)

        plsc.subcore_barrier()

        pltpu.sync_copy(acc_sh.at[pl.ds(ro, RPT)],
                        acc_out.at[cid, pl.ds(ro, RPT)])
        if with_counts:
            pltpu.sync_copy(cnt_sh.at[pl.ds(ro, RPT)],
                            cnt_out.at[cid, pl.ds(ro, RPT)])

    return pl.kernel(body, out_type=out_type, mesh=mesh,
                     scratch_types=scratch,
                     compiler_params=pltpu.CompilerParams(
                         use_tc_tiling_on_sc=False))


def _tc_pre(x, Wl, Wr):
    n, _ = x.shape

    def body(x_ref, wl_ref, wr_ref, ypad_ref, z_ref):
        xv = x_ref[...]
        y = jnp.dot(xv, wl_ref[...], preferred_element_type=jnp.float32)
        zeros = jnp.zeros((N_PAD - n, DH_C), jnp.float32)
        for c in range(NC):
            ypad_ref[c, 0:n, :] = y[:, c * DH_C:(c + 1) * DH_C]
            ypad_ref[c, n:N_PAD, :] = zeros
        z_ref[...] = jnp.dot(xv, wr_ref[...],
                             preferred_element_type=jnp.float32)

    return pl.pallas_call(
        body,
        out_shape=[jax.ShapeDtypeStruct((NC, N_PAD, DH_C), jnp.float32),
                   jax.ShapeDtypeStruct((n, D_H), jnp.float32)],
    )(x, Wl, Wr)


def _mean_ln_silu(acc_ref, cnt_ref, z_ref, bl_ref, g_ref, be_ref):
    agg = jnp.concatenate(
        [acc_ref[c, 0:N, :] for c in range(NC)], axis=1)
    c = (cnt_ref[0, 0:N, 0:1] + cnt_ref[1, 0:N, 0:1])
    mean = agg / jnp.maximum(c, 1.0)
    t = mean + bl_ref[...] + z_ref[...]
    mu = jnp.mean(t, axis=-1, keepdims=True)
    var = jnp.mean((t - mu) ** 2, axis=-1, keepdims=True)
    h = (t - mu) / jnp.sqrt(var + 1e-5) * g_ref[...] + be_ref[...]
    return h * jax.nn.sigmoid(h)


def _tc_mid(acc, cnt, z, bl, g, be, Wl_next, Wr_next):
    def body(acc_ref, cnt_ref, z_ref, bl_ref, g_ref, be_ref, wl_ref, wr_ref,
             ypad_ref, znext_ref):
        h = _mean_ln_silu(acc_ref, cnt_ref, z_ref, bl_ref, g_ref, be_ref)
        y = jnp.dot(h, wl_ref[...], preferred_element_type=jnp.float32)
        zeros = jnp.zeros((N_PAD - N, DH_C), jnp.float32)
        for c in range(NC):
            ypad_ref[c, 0:N, :] = y[:, c * DH_C:(c + 1) * DH_C]
            ypad_ref[c, N:N_PAD, :] = zeros
        znext_ref[...] = jnp.dot(h, wr_ref[...],
                                 preferred_element_type=jnp.float32)

    return pl.pallas_call(
        body,
        out_shape=[jax.ShapeDtypeStruct((NC, N_PAD, DH_C), jnp.float32),
                   jax.ShapeDtypeStruct((N, D_H), jnp.float32)],
    )(acc, cnt, z, bl, g, be, Wl_next, Wr_next)


def _tc_post(acc, cnt, z, bl, g, be, Wm1, bm1, Wm2, bm2):
    def body(acc_ref, cnt_ref, z_ref, bl_ref, g_ref, be_ref,
             wm1_ref, bm1_ref, wm2_ref, bm2_ref, out_ref):
        h = _mean_ln_silu(acc_ref, cnt_ref, z_ref, bl_ref, g_ref, be_ref)
        m = jnp.dot(h, wm1_ref[...], preferred_element_type=jnp.float32)
        m = jnp.maximum(m + bm1_ref[...], 0.0)
        out_ref[...] = (jnp.dot(m, wm2_ref[...],
                                preferred_element_type=jnp.float32)
                        + bm2_ref[...])

    return pl.pallas_call(
        body,
        out_shape=jax.ShapeDtypeStruct((N, 1), jnp.float32),
    )(acc, cnt, z, bl, g, be, Wm1, bm1, Wm2, bm2)


@jax.jit
def kernel(x, edge_index, W0l, b0l, W0r, g0, be0, W1l, b1l, W1r, g1, be1,
           Wm1, bm1, Wm2, bm2):
    src = edge_index[0]
    dst = edge_index[1]
    pad = E_PAD - E
    src_p = jnp.concatenate([src, jnp.full((pad,), N, jnp.int32)])
    dst_p = jnp.concatenate([dst, jnp.full((pad,), N, jnp.int32)])
    src2d = src_p.reshape(NBLK_TOT, BLK)
    dst2d = dst_p.reshape(NBLK_TOT, BLK)

    z32 = jnp.zeros((N_PAD, DH_C), jnp.float32)
    z16 = jnp.zeros((N_PAD, CNT_W), jnp.float32)
    ones = jnp.ones((BLK, CNT_W), jnp.float32)

    b0l_r = b0l.reshape(1, D_H)
    g0_r = g0.reshape(1, D_H)
    be0_r = be0.reshape(1, D_H)
    b1l_r = b1l.reshape(1, D_H)
    g1_r = g1.reshape(1, D_H)
    be1_r = be1.reshape(1, D_H)
    bm1_r = bm1.reshape(1, MLP_H)
    bm2_r = bm2.reshape(1, 1)

    agg0_fn = _sc_aggregate(with_counts=True)
    agg1_fn = _sc_aggregate(with_counts=False)

    y0_pad, z0 = _tc_pre(x, W0l, W0r)
    acc0, cnt = agg0_fn(y0_pad, src2d, dst2d, z32, z16, ones)
    y1_pad, z1 = _tc_mid(acc0, cnt, z0, b0l_r, g0_r, be0_r, W1l, W1r)

    acc1, = agg1_fn(y1_pad, src2d, dst2d, z32, z16, ones)

    return _tc_post(acc1, cnt, z1, b1l_r, g1_r, be1_r,
                    Wm1, bm1_r, Wm2, bm2_r)


# speedup vs baseline: 11.4144x; 1.0170x over previous
import functools

import jax
import jax.numpy as jnp
from jax import lax
from jax.experimental import pallas as pl
from jax.experimental.pallas import tpu as pltpu
from jax.experimental.pallas import tpu_sc as plsc

N = 10000
E = 320000
D_IN = 128
D_H = 64
MLP_H = 128

NC = 2
NS = 16
DH_C = D_H // NC
BLK = 128
N_PAD = 10016
RPT = N_PAD // NS
BPT = 160
HALF = BPT // 2
E_PAD = NS * BPT * BLK
NBLK_TOT = E_PAD // BLK
CNT_W = 16
NBUF = 2
K = 2
SPT = BPT // K
SHALF = SPT // 2


def _sc_aggregate(with_counts):
    mesh = plsc.VectorSubcoreMesh(core_axis_name="c", subcore_axis_name="s")
    out_type = [jax.ShapeDtypeStruct((NC, N_PAD, DH_C), jnp.float32)]
    scratch = [
        pltpu.VMEM((SPT, K * BLK), jnp.int32),
        pltpu.VMEM((SPT, K * BLK), jnp.int32),
        [pltpu.VMEM((K * BLK, DH_C), jnp.float32) for _ in range(NBUF)],
        [pltpu.SemaphoreType.DMA for _ in range(NBUF)],
        [pltpu.SemaphoreType.DMA for _ in range(NBUF)],
        pltpu.VMEM_SHARED((N_PAD, DH_C), jnp.float32),
        pltpu.VMEM_SHARED((N_PAD, DH_C), jnp.float32),
    ]
    if with_counts:
        out_type.append(
            jax.ShapeDtypeStruct((NC, N_PAD, CNT_W), jnp.float32))
        scratch += [
            pltpu.VMEM((K * BLK, CNT_W), jnp.float32),
            pltpu.VMEM_SHARED((N_PAD, CNT_W), jnp.float32),
            [pltpu.SemaphoreType.DMA for _ in range(NBUF)],
        ]

    def body(y_hbm, src_hbm, dst_hbm, z32_hbm, z16_hbm, ones_hbm,
             acc_out, *rest):
        it = iter(rest)
        cnt_out = next(it) if with_counts else None
        src_v = next(it)
        dst_v = next(it)
        rows = next(it)
        gsem = next(it)
        ssem = next(it)
        acc_sh = next(it)
        y_sh = next(it)
        if with_counts:
            ones_v = next(it)
            cnt_sh = next(it)
            csem = next(it)
        cid = lax.axis_index("c")
        sid = lax.axis_index("s")

        ro = pl.multiple_of(sid * RPT, RPT)
        pltpu.sync_copy(y_hbm.at[cid, pl.ds(ro, RPT)], y_sh.at[pl.ds(ro, RPT)])
        pltpu.sync_copy(z32_hbm.at[pl.ds(ro, RPT)], acc_sh.at[pl.ds(ro, RPT)])
        if with_counts:
            pltpu.sync_copy(z16_hbm.at[pl.ds(ro, RPT)],
                            cnt_sh.at[pl.ds(ro, RPT)])
            pltpu.sync_copy(ones_hbm, ones_v)
        bbase = pl.multiple_of(sid * SPT, SPT)
        pltpu.sync_copy(src_hbm.at[pl.ds(bbase, SPT)], src_v)
        pltpu.sync_copy(dst_hbm.at[pl.ds(bbase, SPT)], dst_v)
        plsc.subcore_barrier()

        def start_gather(sb, j):
            pltpu.async_copy(y_sh.at[src_v.at[sb]], rows[j], gsem[j])

        def wait_gather(j):
            pltpu.make_async_copy(y_sh.at[src_v.at[0]], rows[j],
                                  gsem[j]).wait()

        def run_phase(base, count_core):
            do_cnt = with_counts and count_core is not None
            if do_cnt:
                on = cid == count_core

            def start_scatter(sb, j):
                idx = dst_v.at[sb]
                pltpu.async_copy(rows[j], acc_sh.at[idx], ssem[j], add=True)
                if do_cnt:
                    @pl.when(on)
                    def _():
                        pltpu.async_copy(ones_v, cnt_sh.at[idx],
                                         csem[j], add=True)

            def wait_scatter(j):
                idx0 = dst_v.at[0]
                pltpu.make_async_copy(rows[j], acc_sh.at[idx0],
                                      ssem[j]).wait()
                if do_cnt:
                    @pl.when(on)
                    def _():
                        pltpu.make_async_copy(ones_v, cnt_sh.at[idx0],
                                              csem[j]).wait()

            for j in range(NBUF):
                start_gather(base + j, j)

            def step(k, carry):
                b = base + k * NBUF
                for j in range(NBUF):
                    wait_gather(j)
                    start_scatter(b + j, j)
                for j in range(NBUF):
                    wait_scatter(j)
                    start_gather(b + NBUF + j, j)
                return carry

            lax.fori_loop(0, SHALF // NBUF - 1, step, 0)

            tail = base + SHALF - NBUF
            for j in range(NBUF):
                wait_gather(j)
                start_scatter(tail + j, j)
            for j in range(NBUF):
                wait_scatter(j)

        run_phase(0, 0 if with_counts else None)
        run_phase(SHALF, 1 if with_counts else None)

        plsc.subcore_barrier()

        pltpu.sync_copy(acc_sh.at[pl.ds(ro, RPT)],
                        acc_out.at[cid, pl.ds(ro, RPT)])
        if with_counts:
            pltpu.sync_copy(cnt_sh.at[pl.ds(ro, RPT)],
                            cnt_out.at[cid, pl.ds(ro, RPT)])

    return pl.kernel(body, out_type=out_type, mesh=mesh,
                     scratch_types=scratch,
                     compiler_params=pltpu.CompilerParams(
                         use_tc_tiling_on_sc=False))


def _tc_pre(x, Wl, Wr):
    n, _ = x.shape

    def body(x_ref, wl_ref, wr_ref, ypad_ref, z_ref):
        xv = x_ref[...]
        y = jnp.dot(xv, wl_ref[...], preferred_element_type=jnp.float32)
        zeros = jnp.zeros((N_PAD - n, DH_C), jnp.float32)
        for c in range(NC):
            ypad_ref[c, 0:n, :] = y[:, c * DH_C:(c + 1) * DH_C]
            ypad_ref[c, n:N_PAD, :] = zeros
        z_ref[...] = jnp.dot(xv, wr_ref[...],
                             preferred_element_type=jnp.float32)

    return pl.pallas_call(
        body,
        out_shape=[jax.ShapeDtypeStruct((NC, N_PAD, DH_C), jnp.float32),
                   jax.ShapeDtypeStruct((n, D_H), jnp.float32)],
    )(x, Wl, Wr)


def _mean_ln_silu(acc_ref, cnt_ref, z_ref, bl_ref, g_ref, be_ref):
    agg = jnp.concatenate(
        [acc_ref[c, 0:N, :] for c in range(NC)], axis=1)
    c = (cnt_ref[0, 0:N, 0:1] + cnt_ref[1, 0:N, 0:1])
    mean = agg / jnp.maximum(c, 1.0)
    t = mean + bl_ref[...] + z_ref[...]
    mu = jnp.mean(t, axis=-1, keepdims=True)
    var = jnp.mean((t - mu) ** 2, axis=-1, keepdims=True)
    h = (t - mu) / jnp.sqrt(var + 1e-5) * g_ref[...] + be_ref[...]
    return h * jax.nn.sigmoid(h)


def _tc_mid(acc, cnt, z, bl, g, be, Wl_next, Wr_next):
    def body(acc_ref, cnt_ref, z_ref, bl_ref, g_ref, be_ref, wl_ref, wr_ref,
             ypad_ref, znext_ref):
        h = _mean_ln_silu(acc_ref, cnt_ref, z_ref, bl_ref, g_ref, be_ref)
        y = jnp.dot(h, wl_ref[...], preferred_element_type=jnp.float32)
        zeros = jnp.zeros((N_PAD - N, DH_C), jnp.float32)
        for c in range(NC):
            ypad_ref[c, 0:N, :] = y[:, c * DH_C:(c + 1) * DH_C]
            ypad_ref[c, N:N_PAD, :] = zeros
        znext_ref[...] = jnp.dot(h, wr_ref[...],
                                 preferred_element_type=jnp.float32)

    return pl.pallas_call(
        body,
        out_shape=[jax.ShapeDtypeStruct((NC, N_PAD, DH_C), jnp.float32),
                   jax.ShapeDtypeStruct((N, D_H), jnp.float32)],
    )(acc, cnt, z, bl, g, be, Wl_next, Wr_next)


def _tc_post(acc, cnt, z, bl, g, be, Wm1, bm1, Wm2, bm2):
    def body(acc_ref, cnt_ref, z_ref, bl_ref, g_ref, be_ref,
             wm1_ref, bm1_ref, wm2_ref, bm2_ref, out_ref):
        h = _mean_ln_silu(acc_ref, cnt_ref, z_ref, bl_ref, g_ref, be_ref)
        m = jnp.dot(h, wm1_ref[...], preferred_element_type=jnp.float32)
        m = jnp.maximum(m + bm1_ref[...], 0.0)
        out_ref[...] = (jnp.dot(m, wm2_ref[...],
                                preferred_element_type=jnp.float32)
                        + bm2_ref[...])

    return pl.pallas_call(
        body,
        out_shape=jax.ShapeDtypeStruct((N, 1), jnp.float32),
    )(acc, cnt, z, bl, g, be, Wm1, bm1, Wm2, bm2)


@jax.jit
def kernel(x, edge_index, W0l, b0l, W0r, g0, be0, W1l, b1l, W1r, g1, be1,
           Wm1, bm1, Wm2, bm2):
    src = edge_index[0]
    dst = edge_index[1]
    pad = E_PAD - E
    src_p = jnp.concatenate([src, jnp.full((pad,), N, jnp.int32)])
    dst_p = jnp.concatenate([dst, jnp.full((pad,), N, jnp.int32)])
    src2d = src_p.reshape(NBLK_TOT // K, K * BLK)
    dst2d = dst_p.reshape(NBLK_TOT // K, K * BLK)

    z32 = jnp.zeros((N_PAD, DH_C), jnp.float32)
    z16 = jnp.zeros((N_PAD, CNT_W), jnp.float32)
    ones = jnp.ones((K * BLK, CNT_W), jnp.float32)

    b0l_r = b0l.reshape(1, D_H)
    g0_r = g0.reshape(1, D_H)
    be0_r = be0.reshape(1, D_H)
    b1l_r = b1l.reshape(1, D_H)
    g1_r = g1.reshape(1, D_H)
    be1_r = be1.reshape(1, D_H)
    bm1_r = bm1.reshape(1, MLP_H)
    bm2_r = bm2.reshape(1, 1)

    agg0_fn = _sc_aggregate(with_counts=True)
    agg1_fn = _sc_aggregate(with_counts=False)

    y0_pad, z0 = _tc_pre(x, W0l, W0r)
    acc0, cnt = agg0_fn(y0_pad, src2d, dst2d, z32, z16, ones)
    y1_pad, z1 = _tc_mid(acc0, cnt, z0, b0l_r, g0_r, be0_r, W1l, W1r)

    acc1, = agg1_fn(y1_pad, src2d, dst2d, z32, z16, ones)

    return _tc_post(acc1, cnt, z1, b1l_r, g1_r, be1_r,
                    Wm1, bm1_r, Wm2, bm2_r)


# speedup vs baseline: 12.0190x; 1.0530x over previous
import functools

import jax
import jax.numpy as jnp
from jax import lax
from jax.experimental import pallas as pl
from jax.experimental.pallas import tpu as pltpu
from jax.experimental.pallas import tpu_sc as plsc

N = 10000
E = 320000
D_IN = 128
D_H = 64
MLP_H = 128

NC = 2
NS = 16
DH_C = D_H // NC
BLK = 128
N_PAD = 10016
RPT = N_PAD // NS
BPT = 160
HALF = BPT // 2
E_PAD = NS * BPT * BLK
NBLK_TOT = E_PAD // BLK
NROW_REAL = E // 256
NROW_PAD = 30
CNT_W = 16
NBUF = 2
K = 2
SPT = BPT // K
SHALF = SPT // 2


def _sc_aggregate(with_counts):
    mesh = plsc.VectorSubcoreMesh(core_axis_name="c", subcore_axis_name="s")
    out_type = [jax.ShapeDtypeStruct((NC, N_PAD, DH_C), jnp.float32)]
    scratch = [
        pltpu.VMEM((SPT, K * BLK), jnp.int32),
        pltpu.VMEM((SPT, K * BLK), jnp.int32),
        [pltpu.VMEM((K * BLK, DH_C), jnp.float32) for _ in range(NBUF)],
        [pltpu.SemaphoreType.DMA for _ in range(NBUF)],
        [pltpu.SemaphoreType.DMA for _ in range(NBUF)],
        pltpu.VMEM_SHARED((N_PAD, DH_C), jnp.float32),
        pltpu.VMEM_SHARED((N_PAD, DH_C), jnp.float32),
    ]
    if with_counts:
        out_type.append(
            jax.ShapeDtypeStruct((NC, N_PAD, CNT_W), jnp.float32))
        scratch += [
            pltpu.VMEM((K * BLK, CNT_W), jnp.float32),
            pltpu.VMEM_SHARED((N_PAD, CNT_W), jnp.float32),
            [pltpu.SemaphoreType.DMA for _ in range(NBUF)],
        ]

    def body(y_hbm, ei_hbm, pads_hbm, padd_hbm, z32_hbm, z16_hbm, ones_hbm,
             acc_out, *rest):
        it = iter(rest)
        cnt_out = next(it) if with_counts else None
        src_v = next(it)
        dst_v = next(it)
        rows = next(it)
        gsem = next(it)
        ssem = next(it)
        acc_sh = next(it)
        y_sh = next(it)
        if with_counts:
            ones_v = next(it)
            cnt_sh = next(it)
            csem = next(it)
        cid = lax.axis_index("c")
        sid = lax.axis_index("s")

        ro = pl.multiple_of(sid * RPT, RPT)
        pltpu.sync_copy(y_hbm.at[cid, pl.ds(ro, RPT)], y_sh.at[pl.ds(ro, RPT)])
        pltpu.sync_copy(z32_hbm, acc_sh.at[pl.ds(ro, RPT)])
        if with_counts:
            pltpu.sync_copy(z16_hbm, cnt_sh.at[pl.ds(ro, RPT)])
            pltpu.sync_copy(ones_hbm, ones_v)
        bbase = pl.multiple_of(sid * SPT, SPT)
        real = NROW_REAL - (NS - 1) * SPT

        @pl.when(sid < NS - 1)
        def _():
            pltpu.sync_copy(ei_hbm.at[0, pl.ds(bbase, SPT)], src_v)
            pltpu.sync_copy(ei_hbm.at[1, pl.ds(bbase, SPT)], dst_v)

        @pl.when(sid == NS - 1)
        def _():
            lastb = pl.multiple_of((NS - 1) * SPT, SPT)
            pltpu.sync_copy(ei_hbm.at[0, pl.ds(lastb, real)],
                            src_v.at[pl.ds(0, real)])
            pltpu.sync_copy(ei_hbm.at[1, pl.ds(lastb, real)],
                            dst_v.at[pl.ds(0, real)])
            pltpu.sync_copy(pads_hbm, src_v.at[pl.ds(real, NROW_PAD)])
            pltpu.sync_copy(padd_hbm, dst_v.at[pl.ds(real, NROW_PAD)])

        plsc.subcore_barrier()

        def start_gather(sb, j):
            pltpu.async_copy(y_sh.at[src_v.at[sb]], rows[j], gsem[j])

        def wait_gather(j):
            pltpu.make_async_copy(y_sh.at[src_v.at[0]], rows[j],
                                  gsem[j]).wait()

        def run_phase(base, count_core):
            do_cnt = with_counts and count_core is not None
            if do_cnt:
                on = cid == count_core

            def start_scatter(sb, j):
                idx = dst_v.at[sb]
                pltpu.async_copy(rows[j], acc_sh.at[idx], ssem[j], add=True)
                if do_cnt:
                    @pl.when(on)
                    def _():
                        pltpu.async_copy(ones_v, cnt_sh.at[idx],
                                         csem[j], add=True)

            def wait_scatter(j):
                idx0 = dst_v.at[0]
                pltpu.make_async_copy(rows[j], acc_sh.at[idx0],
                                      ssem[j]).wait()
                if do_cnt:
                    @pl.when(on)
                    def _():
                        pltpu.make_async_copy(ones_v, cnt_sh.at[idx0],
                                              csem[j]).wait()

            for j in range(NBUF):
                start_gather(base + j, j)

            def step(k, carry):
                b = base + k * NBUF
                for j in range(NBUF):
                    wait_gather(j)
                    start_scatter(b + j, j)
                for j in range(NBUF):
                    wait_scatter(j)
                    start_gather(b + NBUF + j, j)
                return carry

            lax.fori_loop(0, SHALF // NBUF - 1, step, 0)

            tail = base + SHALF - NBUF
            for j in range(NBUF):
                wait_gather(j)
                start_scatter(tail + j, j)
            for j in range(NBUF):
                wait_scatter(j)

        run_phase(0, 0 if with_counts else None)
        run_phase(SHALF, 1 if with_counts else None)

        plsc.subcore_barrier()

        pltpu.sync_copy(acc_sh.at[pl.ds(ro, RPT)],
                        acc_out.at[cid, pl.ds(ro, RPT)])
        if with_counts:
            pltpu.sync_copy(cnt_sh.at[pl.ds(ro, RPT)],
                            cnt_out.at[cid, pl.ds(ro, RPT)])

    return pl.kernel(body, out_type=out_type, mesh=mesh,
                     scratch_types=scratch,
                     compiler_params=pltpu.CompilerParams(
                         use_tc_tiling_on_sc=False))


def _tc_pre(x, Wl, Wr):
    n, _ = x.shape

    def body(x_ref, wl_ref, wr_ref, ypad_ref, z_ref):
        xv = x_ref[...]
        y = jnp.dot(xv, wl_ref[...], preferred_element_type=jnp.float32)
        zeros = jnp.zeros((N_PAD - n, DH_C), jnp.float32)
        for c in range(NC):
            ypad_ref[c, 0:n, :] = y[:, c * DH_C:(c + 1) * DH_C]
            ypad_ref[c, n:N_PAD, :] = zeros
        z_ref[...] = jnp.dot(xv, wr_ref[...],
                             preferred_element_type=jnp.float32)

    return pl.pallas_call(
        body,
        out_shape=[jax.ShapeDtypeStruct((NC, N_PAD, DH_C), jnp.float32),
                   jax.ShapeDtypeStruct((n, D_H), jnp.float32)],
    )(x, Wl, Wr)


def _mean_ln_silu(acc_ref, cnt_ref, z_ref, bl_ref, g_ref, be_ref):
    agg = jnp.concatenate(
        [acc_ref[c, 0:N, :] for c in range(NC)], axis=1)
    c = (cnt_ref[0, 0:N, 0:1] + cnt_ref[1, 0:N, 0:1])
    mean = agg / jnp.maximum(c, 1.0)
    t = mean + bl_ref[...] + z_ref[...]
    mu = jnp.mean(t, axis=-1, keepdims=True)
    var = jnp.mean((t - mu) ** 2, axis=-1, keepdims=True)
    h = (t - mu) / jnp.sqrt(var + 1e-5) * g_ref[...] + be_ref[...]
    return h * jax.nn.sigmoid(h)


def _tc_mid(acc, cnt, z, bl, g, be, Wl_next, Wr_next):
    def body(acc_ref, cnt_ref, z_ref, bl_ref, g_ref, be_ref, wl_ref, wr_ref,
             ypad_ref, znext_ref):
        h = _mean_ln_silu(acc_ref, cnt_ref, z_ref, bl_ref, g_ref, be_ref)
        y = jnp.dot(h, wl_ref[...], preferred_element_type=jnp.float32)
        zeros = jnp.zeros((N_PAD - N, DH_C), jnp.float32)
        for c in range(NC):
            ypad_ref[c, 0:N, :] = y[:, c * DH_C:(c + 1) * DH_C]
            ypad_ref[c, N:N_PAD, :] = zeros
        znext_ref[...] = jnp.dot(h, wr_ref[...],
                                 preferred_element_type=jnp.float32)

    return pl.pallas_call(
        body,
        out_shape=[jax.ShapeDtypeStruct((NC, N_PAD, DH_C), jnp.float32),
                   jax.ShapeDtypeStruct((N, D_H), jnp.float32)],
    )(acc, cnt, z, bl, g, be, Wl_next, Wr_next)


def _tc_post(acc, cnt, z, bl, g, be, Wm1, bm1, Wm2, bm2):
    def body(acc_ref, cnt_ref, z_ref, bl_ref, g_ref, be_ref,
             wm1_ref, bm1_ref, wm2_ref, bm2_ref, out_ref):
        h = _mean_ln_silu(acc_ref, cnt_ref, z_ref, bl_ref, g_ref, be_ref)
        m = jnp.dot(h, wm1_ref[...], preferred_element_type=jnp.float32)
        m = jnp.maximum(m + bm1_ref[...], 0.0)
        out_ref[...] = (jnp.dot(m, wm2_ref[...],
                                preferred_element_type=jnp.float32)
                        + bm2_ref[...])

    return pl.pallas_call(
        body,
        out_shape=jax.ShapeDtypeStruct((N, 1), jnp.float32),
    )(acc, cnt, z, bl, g, be, Wm1, bm1, Wm2, bm2)


@jax.jit
def kernel(x, edge_index, W0l, b0l, W0r, g0, be0, W1l, b1l, W1r, g1, be1,
           Wm1, bm1, Wm2, bm2):
    ei3 = edge_index.reshape(2, NROW_REAL, K * BLK)
    padidx = (N + jnp.arange(NROW_PAD * K * BLK, dtype=jnp.int32)
              % (N_PAD - N)).reshape(NROW_PAD, K * BLK)

    z32 = jnp.zeros((RPT, DH_C), jnp.float32)
    z16 = jnp.zeros((RPT, CNT_W), jnp.float32)
    ones = jnp.ones((K * BLK, CNT_W), jnp.float32)

    b0l_r = b0l.reshape(1, D_H)
    g0_r = g0.reshape(1, D_H)
    be0_r = be0.reshape(1, D_H)
    b1l_r = b1l.reshape(1, D_H)
    g1_r = g1.reshape(1, D_H)
    be1_r = be1.reshape(1, D_H)
    bm1_r = bm1.reshape(1, MLP_H)
    bm2_r = bm2.reshape(1, 1)

    agg0_fn = _sc_aggregate(with_counts=True)
    agg1_fn = _sc_aggregate(with_counts=False)

    y0_pad, z0 = _tc_pre(x, W0l, W0r)
    acc0, cnt = agg0_fn(y0_pad, ei3, padidx, padidx, z32, z16, ones)
    y1_pad, z1 = _tc_mid(acc0, cnt, z0, b0l_r, g0_r, be0_r, W1l, W1r)

    acc1, = agg1_fn(y1_pad, ei3, padidx, padidx, z32, z16, ones)

    return _tc_post(acc1, cnt, z1, b1l_r, g1_r, be1_r,
                    Wm1, bm1_r, Wm2, bm2_r)


# speedup vs baseline: 12.0372x; 1.0015x over previous
import jax
import jax.numpy as jnp
from jax import lax
from jax.experimental import pallas as pl
from jax.experimental.pallas import tpu as pltpu
from jax.experimental.pallas import tpu_sc as plsc

N = 10000
E = 320000
D_IN = 128
D_H = 64
MLP_H = 128

NC = 2
NS = 16
DH_C = D_H // NC
BLK = 128
N_PAD = 10016
RPT = N_PAD // NS
BPT = 160
HALF = BPT // 2
E_PAD = NS * BPT * BLK
NBLK_TOT = E_PAD // BLK
NROW_REAL = E // 256
NROW_PAD = 30
CNT_W = 16
NBUF = 2
K = 2
SPT = BPT // K
SHALF = SPT // 2


def _sc_aggregate(with_counts):
    mesh = plsc.VectorSubcoreMesh(core_axis_name="c", subcore_axis_name="s")
    out_type = [jax.ShapeDtypeStruct((NC, N_PAD, DH_C), jnp.float32)]
    scratch = [
        pltpu.VMEM((SPT, K * BLK), jnp.int32),
        pltpu.VMEM((SPT, K * BLK), jnp.int32),
        [pltpu.VMEM((K * BLK, DH_C), jnp.float32) for _ in range(NBUF)],
        [pltpu.SemaphoreType.DMA for _ in range(NBUF)],
        [pltpu.SemaphoreType.DMA for _ in range(NBUF)],
        pltpu.VMEM_SHARED((N_PAD, DH_C), jnp.float32),
        pltpu.VMEM_SHARED((N_PAD, DH_C), jnp.float32),
    ]
    if with_counts:
        out_type.append(
            jax.ShapeDtypeStruct((NC, N_PAD, CNT_W), jnp.float32))
        scratch += [
            pltpu.VMEM((K * BLK, CNT_W), jnp.float32),
            pltpu.VMEM_SHARED((N_PAD, CNT_W), jnp.float32),
            [pltpu.SemaphoreType.DMA for _ in range(NBUF)],
        ]

    def body(y_hbm, ei_hbm, pads_hbm, padd_hbm, z32_hbm, z16_hbm, ones_hbm,
             acc_out, *rest):
        it = iter(rest)
        cnt_out = next(it) if with_counts else None
        src_v = next(it)
        dst_v = next(it)
        rows = next(it)
        gsem = next(it)
        ssem = next(it)
        acc_sh = next(it)
        y_sh = next(it)
        if with_counts:
            ones_v = next(it)
            cnt_sh = next(it)
            csem = next(it)
        cid = lax.axis_index("c")
        sid = lax.axis_index("s")

        ro = pl.multiple_of(sid * RPT, RPT)
        pltpu.sync_copy(y_hbm.at[cid, pl.ds(ro, RPT)], y_sh.at[pl.ds(ro, RPT)])
        pltpu.sync_copy(z32_hbm, acc_sh.at[pl.ds(ro, RPT)])
        if with_counts:
            pltpu.sync_copy(z16_hbm, cnt_sh.at[pl.ds(ro, RPT)])
            pltpu.sync_copy(ones_hbm, ones_v)
        bbase = pl.multiple_of(sid * SPT, SPT)
        real = NROW_REAL - (NS - 1) * SPT

        @pl.when(sid < NS - 1)
        def _():
            pltpu.sync_copy(ei_hbm.at[0, pl.ds(bbase, SPT)], src_v)
            pltpu.sync_copy(ei_hbm.at[1, pl.ds(bbase, SPT)], dst_v)

        @pl.when(sid == NS - 1)
        def _():
            lastb = pl.multiple_of((NS - 1) * SPT, SPT)
            pltpu.sync_copy(ei_hbm.at[0, pl.ds(lastb, real)],
                            src_v.at[pl.ds(0, real)])
            pltpu.sync_copy(ei_hbm.at[1, pl.ds(lastb, real)],
                            dst_v.at[pl.ds(0, real)])
            pltpu.sync_copy(pads_hbm, src_v.at[pl.ds(real, NROW_PAD)])
            pltpu.sync_copy(padd_hbm, dst_v.at[pl.ds(real, NROW_PAD)])

        plsc.subcore_barrier()

        def start_gather(sb, j):
            pltpu.async_copy(y_sh.at[src_v.at[sb]], rows[j], gsem[j])

        def wait_gather(j):
            pltpu.make_async_copy(y_sh.at[src_v.at[0]], rows[j],
                                  gsem[j]).wait()

        def run_phase(base, count_core):
            do_cnt = with_counts and count_core is not None
            if do_cnt:
                on = cid == count_core

            def start_scatter(sb, j):
                idx = dst_v.at[sb]
                pltpu.async_copy(rows[j], acc_sh.at[idx], ssem[j], add=True)
                if do_cnt:
                    @pl.when(on)
                    def _():
                        pltpu.async_copy(ones_v, cnt_sh.at[idx],
                                         csem[j], add=True)

            def wait_scatter(j):
                idx0 = dst_v.at[0]
                pltpu.make_async_copy(rows[j], acc_sh.at[idx0],
                                      ssem[j]).wait()
                if do_cnt:
                    @pl.when(on)
                    def _():
                        pltpu.make_async_copy(ones_v, cnt_sh.at[idx0],
                                              csem[j]).wait()

            for j in range(NBUF):
                start_gather(base + j, j)

            def step(k, carry):
                b = base + k * NBUF
                for j in range(NBUF):
                    wait_gather(j)
                    start_scatter(b + j, j)
                for j in range(NBUF):
                    wait_scatter(j)
                    start_gather(b + NBUF + j, j)
                return carry

            lax.fori_loop(0, SHALF // NBUF - 1, step, 0)

            tail = base + SHALF - NBUF
            for j in range(NBUF):
                wait_gather(j)
                start_scatter(tail + j, j)
            for j in range(NBUF):
                wait_scatter(j)

        run_phase(0, 0 if with_counts else None)
        run_phase(SHALF, 1 if with_counts else None)

        plsc.subcore_barrier()

        pltpu.sync_copy(acc_sh.at[pl.ds(ro, RPT)],
                        acc_out.at[cid, pl.ds(ro, RPT)])
        if with_counts:
            pltpu.sync_copy(cnt_sh.at[pl.ds(ro, RPT)],
                            cnt_out.at[cid, pl.ds(ro, RPT)])

    return pl.kernel(body, out_type=out_type, mesh=mesh,
                     scratch_types=scratch,
                     compiler_params=pltpu.CompilerParams(
                         use_tc_tiling_on_sc=False))


def _tc_pre(x, Wl, Wr):
    n, _ = x.shape

    def body(x_ref, wl_ref, wr_ref, ypad_ref, z_ref):
        xv = x_ref[...]
        y = jnp.dot(xv, wl_ref[...], preferred_element_type=jnp.float32)
        zeros = jnp.zeros((N_PAD - n, DH_C), jnp.float32)
        for c in range(NC):
            ypad_ref[c, 0:n, :] = y[:, c * DH_C:(c + 1) * DH_C]
            ypad_ref[c, n:N_PAD, :] = zeros
        z_ref[...] = jnp.dot(xv, wr_ref[...],
                             preferred_element_type=jnp.float32)

    return pl.pallas_call(
        body,
        out_shape=[jax.ShapeDtypeStruct((NC, N_PAD, DH_C), jnp.float32),
                   jax.ShapeDtypeStruct((n, D_H), jnp.float32)],
    )(x, Wl, Wr)


def _mean_ln_silu(acc_ref, cnt_ref, z_ref, bl_ref, g_ref, be_ref):
    agg = jnp.concatenate(
        [acc_ref[c, 0:N, :] for c in range(NC)], axis=1)
    c = (cnt_ref[0, 0:N, 0:1] + cnt_ref[1, 0:N, 0:1])
    mean = agg / jnp.maximum(c, 1.0)
    t = mean + bl_ref[...] + z_ref[...]
    mu = jnp.mean(t, axis=-1, keepdims=True)
    var = jnp.mean((t - mu) ** 2, axis=-1, keepdims=True)
    h = (t - mu) / jnp.sqrt(var + 1e-5) * g_ref[...] + be_ref[...]
    return h * jax.nn.sigmoid(h)


def _tc_mid(acc, cnt, z, bl, g, be, Wl_next, Wr_next):
    def body(acc_ref, cnt_ref, z_ref, bl_ref, g_ref, be_ref, wl_ref, wr_ref,
             ypad_ref, znext_ref):
        h = _mean_ln_silu(acc_ref, cnt_ref, z_ref, bl_ref, g_ref, be_ref)
        y = jnp.dot(h, wl_ref[...], preferred_element_type=jnp.float32)
        zeros = jnp.zeros((N_PAD - N, DH_C), jnp.float32)
        for c in range(NC):
            ypad_ref[c, 0:N, :] = y[:, c * DH_C:(c + 1) * DH_C]
            ypad_ref[c, N:N_PAD, :] = zeros
        znext_ref[...] = jnp.dot(h, wr_ref[...],
                                 preferred_element_type=jnp.float32)

    return pl.pallas_call(
        body,
        out_shape=[jax.ShapeDtypeStruct((NC, N_PAD, DH_C), jnp.float32),
                   jax.ShapeDtypeStruct((N, D_H), jnp.float32)],
    )(acc, cnt, z, bl, g, be, Wl_next, Wr_next)


def _tc_post(acc, cnt, z, bl, g, be, Wm1, bm1, Wm2, bm2):
    def body(acc_ref, cnt_ref, z_ref, bl_ref, g_ref, be_ref,
             wm1_ref, bm1_ref, wm2_ref, bm2_ref, out_ref):
        h = _mean_ln_silu(acc_ref, cnt_ref, z_ref, bl_ref, g_ref, be_ref)
        m = jnp.dot(h, wm1_ref[...], preferred_element_type=jnp.float32)
        m = jnp.maximum(m + bm1_ref[...], 0.0)
        out_ref[...] = (jnp.dot(m, wm2_ref[...],
                                preferred_element_type=jnp.float32)
                        + bm2_ref[...])

    return pl.pallas_call(
        body,
        out_shape=jax.ShapeDtypeStruct((N, 1), jnp.float32),
    )(acc, cnt, z, bl, g, be, Wm1, bm1, Wm2, bm2)


@jax.jit
def kernel(x, edge_index, W0l, b0l, W0r, g0, be0, W1l, b1l, W1r, g1, be1,
           Wm1, bm1, Wm2, bm2):
    ei3 = edge_index.reshape(2, NROW_REAL, K * BLK)
    padidx = (N + jnp.arange(NROW_PAD * K * BLK, dtype=jnp.int32)
              % (N_PAD - N)).reshape(NROW_PAD, K * BLK)

    z32 = jnp.zeros((RPT, DH_C), jnp.float32)
    z16 = jnp.zeros((RPT, CNT_W), jnp.float32)
    ones = jnp.ones((K * BLK, CNT_W), jnp.float32)

    b0l_r = b0l.reshape(1, D_H)
    g0_r = g0.reshape(1, D_H)
    be0_r = be0.reshape(1, D_H)
    b1l_r = b1l.reshape(1, D_H)
    g1_r = g1.reshape(1, D_H)
    be1_r = be1.reshape(1, D_H)
    bm1_r = bm1.reshape(1, MLP_H)
    bm2_r = bm2.reshape(1, 1)

    agg0_fn = _sc_aggregate(with_counts=True)
    agg1_fn = _sc_aggregate(with_counts=False)

    y0_pad, z0 = _tc_pre(x, W0l, W0r)
    acc0, cnt = agg0_fn(y0_pad, ei3, padidx, padidx, z32, z16, ones)
    y1_pad, z1 = _tc_mid(acc0, cnt, z0, b0l_r, g0_r, be0_r, W1l, W1r)

    acc1, = agg1_fn(y1_pad, ei3, padidx, padidx, z32, z16, ones)

    return _tc_post(acc1, cnt, z1, b1l_r, g1_r, be1_r,
                    Wm1, bm1_r, Wm2, bm2_r)
